# baseline (speedup 1.0000x reference)
import functools

import jax
import jax.numpy as jnp
from jax import lax
from jax.experimental import pallas as pl
from jax.experimental.pallas import tpu as pltpu
from jax.experimental.pallas import tpu_sc as plsc

_HW = 1024
_NCLS = 150
_TAU = 0.3
_C = 96
_NCORES = 2
_COPY_ROWS = 16 * 1024
_ROWS_PER_W = _COPY_ROWS // 32


def _scan_body(prob_ref, idx_ref, mask_ref):
    p = prob_ref[0]
    m = jnp.max(p, axis=0, keepdims=True)
    s = jnp.sum(p, axis=0, keepdims=True)
    hw_iota = jax.lax.broadcasted_iota(jnp.int32, p.shape, 0)
    idx = jnp.min(jnp.where(p == m, hw_iota, _HW), axis=0, keepdims=True)
    rep = (s * (1.0 / _HW)) > _TAU
    idx_ref[0] = idx
    mask_ref[0] = rep.astype(jnp.float32)


def _sc_copy_body(emb_hbm, out_hbm, buf_v, sem):
    wid = lax.axis_index("s") * _NCORES + lax.axis_index("c")
    base = wid * _ROWS_PER_W
    pltpu.sync_copy(emb_hbm.at[pl.ds(base, _ROWS_PER_W)], buf_v)
    pltpu.sync_copy(buf_v, out_hbm.at[pl.ds(base, _ROWS_PER_W)])


def kernel(emb, prob_map):
    B = emb.shape[0]
    prob_flat = prob_map.reshape(B, _HW, _NCLS)
    emb_flat = emb.reshape(B * _HW, _C)

    idx, mask = pl.pallas_call(
        _scan_body,
        grid=(B,),
        in_specs=[pl.BlockSpec((1, _HW, _NCLS), lambda b: (b, 0, 0))],
        out_specs=[
            pl.BlockSpec((1, 1, _NCLS), lambda b: (b, 0, 0)),
            pl.BlockSpec((1, 1, _NCLS), lambda b: (b, 0, 0)),
        ],
        out_shape=[
            jax.ShapeDtypeStruct((B, 1, _NCLS), jnp.int32),
            jax.ShapeDtypeStruct((B, 1, _NCLS), jnp.float32),
        ],
    )(prob_flat)

    mesh = plsc.VectorSubcoreMesh(core_axis_name="c", subcore_axis_name="s")
    sc_copy = functools.partial(
        pl.kernel,
        mesh=mesh,
        out_type=jax.ShapeDtypeStruct((_COPY_ROWS, _C), jnp.float32),
        scratch_types=[
            pltpu.VMEM((_ROWS_PER_W, _C), jnp.float32),
            pltpu.SemaphoreType.DMA,
        ],
    )(_sc_copy_body)
    copied = sc_copy(emb_flat)

    return (
        jnp.zeros((B, _NCLS, _C), jnp.float32)
        + (idx.astype(jnp.float32) + mask).reshape(B, _NCLS, 1)
        + copied[0, 0]
    )


# speedup vs baseline: 1.3375x; 1.3375x over previous
import jax
import jax.numpy as jnp
from jax.experimental import pallas as pl

_H, _W, _C = 32, 32, 96
_HW = _H * _W
_NCLS = 150
_TAU = 0.3


def _body(prob_ref, emb_ref, out_ref):
  for i in range(2):
    p = prob_ref[i]
    e = emb_ref[i]
    m = jnp.max(p, axis=0, keepdims=True)
    s = jnp.sum(p, axis=0, keepdims=True)
    hw_iota = jax.lax.broadcasted_iota(jnp.int32, p.shape, 0)
    idx = jnp.min(jnp.where(p == m, hw_iota, _HW), axis=0, keepdims=True)
    rep = (s * (1.0 / _HW)) > _TAU
    onehot = ((hw_iota == idx) & rep).astype(jnp.float32)
    out_ref[i] = jax.lax.dot_general(
        onehot, e, (((0,), (0,)), ((), ())),
        preferred_element_type=jnp.float32,
    )


def kernel(emb, prob_map):
    B = emb.shape[0]
    emb_flat = emb.reshape(B, _HW, _C)
    prob_flat = prob_map.reshape(B, _HW, _NCLS)
    out = pl.pallas_call(
        _body,
        grid=(B // 2,),
        in_specs=[
            pl.BlockSpec((2, _HW, _NCLS), lambda b: (b, 0, 0)),
            pl.BlockSpec((2, _HW, _C), lambda b: (b, 0, 0)),
        ],
        out_specs=pl.BlockSpec((2, _NCLS, _C), lambda b: (b, 0, 0)),
        out_shape=jax.ShapeDtypeStruct((B, _NCLS, _C), jnp.float32),
    )(prob_flat, emb_flat)
    return out


# speedup vs baseline: 1.5657x; 1.1707x over previous
import jax
import jax.numpy as jnp
from jax.experimental import pallas as pl

_H, _W, _C = 32, 32, 96
_HW = _H * _W
_NCLS = 150
_TAU = 0.3


def _body(prob_ref, emb_ref, out_ref):
  for i in range(4):
    p = prob_ref[i]
    e = emb_ref[i]
    m = jnp.max(p, axis=0, keepdims=True)
    s = jnp.sum(p, axis=0, keepdims=True)
    hw_iota = jax.lax.broadcasted_iota(jnp.int32, p.shape, 0)
    idx = jnp.min(jnp.where(p == m, hw_iota, _HW), axis=0, keepdims=True)
    rep = (s * (1.0 / _HW)) > _TAU
    onehot = ((hw_iota == idx) & rep).astype(jnp.float32)
    out_ref[i] = jax.lax.dot_general(
        onehot, e, (((0,), (0,)), ((), ())),
        preferred_element_type=jnp.float32,
    )


def kernel(emb, prob_map):
    B = emb.shape[0]
    emb_flat = emb.reshape(B, _HW, _C)
    prob_flat = prob_map.reshape(B, _HW, _NCLS)
    out = pl.pallas_call(
        _body,
        grid=(B // 4,),
        in_specs=[
            pl.BlockSpec((4, _HW, _NCLS), lambda b: (b, 0, 0)),
            pl.BlockSpec((4, _HW, _C), lambda b: (b, 0, 0)),
        ],
        out_specs=pl.BlockSpec((4, _NCLS, _C), lambda b: (b, 0, 0)),
        out_shape=jax.ShapeDtypeStruct((B, _NCLS, _C), jnp.float32),
    )(prob_flat, emb_flat)
    return out


# speedup vs baseline: 1.6459x; 1.0512x over previous
import jax
import jax.numpy as jnp
from jax.experimental import pallas as pl

_H, _W, _C = 32, 32, 96
_HW = _H * _W
_NCLS = 150
_TAU = 0.3


def _body(prob_ref, emb_ref, out_ref):
  for i in range(8):
    p = prob_ref[i]
    e = emb_ref[i]
    m = jnp.max(p, axis=0, keepdims=True)
    s = jnp.sum(p, axis=0, keepdims=True)
    hw_iota = jax.lax.broadcasted_iota(jnp.int32, p.shape, 0)
    idx = jnp.min(jnp.where(p == m, hw_iota, _HW), axis=0, keepdims=True)
    rep = (s * (1.0 / _HW)) > _TAU
    onehot = ((hw_iota == idx) & rep).astype(jnp.float32)
    out_ref[i] = jax.lax.dot_general(
        onehot, e, (((0,), (0,)), ((), ())),
        preferred_element_type=jnp.float32,
    )


def kernel(emb, prob_map):
    B = emb.shape[0]
    emb_flat = emb.reshape(B, _HW, _C)
    prob_flat = prob_map.reshape(B, _HW, _NCLS)
    out = pl.pallas_call(
        _body,
        grid=(B // 8,),
        in_specs=[
            pl.BlockSpec((8, _HW, _NCLS), lambda b: (b, 0, 0)),
            pl.BlockSpec((8, _HW, _C), lambda b: (b, 0, 0)),
        ],
        out_specs=pl.BlockSpec((8, _NCLS, _C), lambda b: (b, 0, 0)),
        out_shape=jax.ShapeDtypeStruct((B, _NCLS, _C), jnp.float32),
    )(prob_flat, emb_flat)
    return out
